# baseline (speedup 1.0000x reference)
import numpy as np

import jax
import jax.numpy as jnp
from jax import lax
from jax.experimental import pallas as pl
from jax.experimental.pallas import tpu as pltpu
from jax.experimental.pallas import tpu_sc as plsc

B = 64
T = 577
D = 768
N = T - 1
NUM_REMAIN = N // 4
OUT_T = NUM_REMAIN + 1

NC, NS = 2, 16
NW = NC * NS
BPW = B // NW
CHA = 80
CHC = 64
L = 16


def _rotl32(x, r):
    return (x << np.uint32(r)) | (x >> np.uint32(32 - r))


def _threefry2x32(k0, k1, x0, x1):
    rotations = ((13, 15, 26, 6), (17, 29, 16, 24))
    ks = (np.uint32(k0), np.uint32(k1),
          np.uint32(k0) ^ np.uint32(k1) ^ np.uint32(0x1BD11BDA))
    x0 = x0 + ks[0]
    x1 = x1 + ks[1]
    with np.errstate(over="ignore"):
        for i in range(5):
            for r in rotations[i % 2]:
                x0 = x0 + x1
                x1 = _rotl32(x1, r)
                x1 = x1 ^ x0
            x0 = x0 + ks[(i + 1) % 3]
            x1 = x1 + ks[(i + 2) % 3] + np.uint32(i + 1)
    return x0, x1


def _fixed_uniform_noise(seed, shape):
    size = int(np.prod(shape))
    o0, o1 = _threefry2x32(0, seed, np.zeros(size, np.uint32),
                           np.arange(size, dtype=np.uint32))
    bits = o0 ^ o1
    floats = (bits >> np.uint32(9)) | np.uint32(0x3F800000)
    return (floats.view(np.float32) - np.float32(1.0)).reshape(shape)


def _index_constants():
    noise = _fixed_uniform_noise(42, (B, N))
    shuffle = np.argsort(noise, axis=-1, kind="stable").astype(np.int32)
    revert = np.argsort(shuffle, axis=-1, kind="stable").astype(np.int32)
    remain = shuffle[:, :NUM_REMAIN]
    masked = shuffle[:, NUM_REMAIN:]

    loc = np.concatenate(
        [np.zeros((B, 1), np.int32), 1 + remain.astype(np.int32)], axis=1)
    idx_a = loc[:, :CHA].reshape(NW, BPW, CHA)
    idx_c = loc[:, CHA:CHA + CHC].reshape(NW, BPW, CHC)
    idx_l = np.repeat(loc[:, CHA + CHC:], L, axis=1).reshape(NW, BPW, L)
    return remain, masked, revert, idx_a, idx_c, idx_l


_REMAIN, _MASKED, _REVERT, _IDX_A, _IDX_C, _IDX_L = _index_constants()


def _gather_kernel(data_hbm, idxa_hbm, idxc_hbm, idxl_hbm, out_hbm, last_hbm,
                   idxa_v, idxc_v, idxl_v, buf_a, buf_c, buf_l, sem_a, sem_c,
                   sem_l):
    wid = lax.axis_index("s") * NC + lax.axis_index("c")
    pltpu.sync_copy(idxa_hbm.at[wid], idxa_v)
    pltpu.sync_copy(idxc_hbm.at[wid], idxc_v)
    pltpu.sync_copy(idxl_hbm.at[wid], idxl_v)

    for j in range(BPW):
        b = wid * BPW + j
        pltpu.async_copy(data_hbm.at[b].at[idxa_v.at[j]], buf_a, sem_a)
        pltpu.async_copy(data_hbm.at[b].at[idxc_v.at[j]], buf_c, sem_c)
        pltpu.async_copy(data_hbm.at[b].at[idxl_v.at[j]], buf_l, sem_l)
        pltpu.make_async_copy(data_hbm.at[b].at[idxa_v.at[j]], buf_a, sem_a).wait()
        pltpu.sync_copy(buf_a, out_hbm.at[b, pl.ds(0, CHA)])
        pltpu.make_async_copy(data_hbm.at[b].at[idxc_v.at[j]], buf_c, sem_c).wait()
        pltpu.sync_copy(buf_c, out_hbm.at[b, pl.ds(CHA, CHC)])
        pltpu.make_async_copy(data_hbm.at[b].at[idxl_v.at[j]], buf_l, sem_l).wait()
        pltpu.sync_copy(buf_l, last_hbm.at[b])


@jax.jit
def _run(data):
    mesh = plsc.VectorSubcoreMesh(core_axis_name="c", subcore_axis_name="s")
    out_main, out_last = pl.kernel(
        _gather_kernel,
        mesh=mesh,
        out_type=[
            jax.ShapeDtypeStruct((B, OUT_T, D), jnp.float32),
            jax.ShapeDtypeStruct((B, L, D), jnp.float32),
        ],
        scratch_types=[
            pltpu.VMEM((BPW, CHA), jnp.int32),
            pltpu.VMEM((BPW, CHC), jnp.int32),
            pltpu.VMEM((BPW, L), jnp.int32),
            pltpu.VMEM((CHA, D), jnp.float32),
            pltpu.VMEM((CHC, D), jnp.float32),
            pltpu.VMEM((L, D), jnp.float32),
            pltpu.SemaphoreType.DMA,
            pltpu.SemaphoreType.DMA,
            pltpu.SemaphoreType.DMA,
        ],
    )(data, jnp.asarray(_IDX_A), jnp.asarray(_IDX_C), jnp.asarray(_IDX_L))

    img_remain = lax.dynamic_update_slice(
        out_main, out_last[:, :1, :], (0, OUT_T - 1, 0))

    remain_idx = jnp.asarray(_REMAIN)
    masked_idx = jnp.asarray(_MASKED)
    revert_idx = jnp.asarray(_REVERT)
    remain_padding_mask = jnp.ones((B, OUT_T), dtype=jnp.float32)
    revert_padding_mask = jnp.ones((B, T), dtype=jnp.float32)
    return (img_remain, remain_idx, masked_idx, revert_idx,
            remain_padding_mask, revert_padding_mask)


def kernel(data):
    return _run(data)


# speedup vs baseline: 1.0827x; 1.0827x over previous
import numpy as np

import jax
import jax.numpy as jnp
from jax import lax
from jax.experimental import pallas as pl
from jax.experimental.pallas import tpu as pltpu
from jax.experimental.pallas import tpu_sc as plsc

B = 64
T = 577
D = 768
N = T - 1
NUM_REMAIN = N // 4
OUT_T = NUM_REMAIN + 1

NC, NS = 2, 16
NW = NC * NS
BPW = B // NW
CHA = 80
CHC = 64
L = 16


def _rotl32(x, r):
    return (x << np.uint32(r)) | (x >> np.uint32(32 - r))


def _threefry2x32(k0, k1, x0, x1):
    rotations = ((13, 15, 26, 6), (17, 29, 16, 24))
    ks = (np.uint32(k0), np.uint32(k1),
          np.uint32(k0) ^ np.uint32(k1) ^ np.uint32(0x1BD11BDA))
    x0 = x0 + ks[0]
    x1 = x1 + ks[1]
    with np.errstate(over="ignore"):
        for i in range(5):
            for r in rotations[i % 2]:
                x0 = x0 + x1
                x1 = _rotl32(x1, r)
                x1 = x1 ^ x0
            x0 = x0 + ks[(i + 1) % 3]
            x1 = x1 + ks[(i + 2) % 3] + np.uint32(i + 1)
    return x0, x1


def _fixed_uniform_noise(seed, shape):
    size = int(np.prod(shape))
    o0, o1 = _threefry2x32(0, seed, np.zeros(size, np.uint32),
                           np.arange(size, dtype=np.uint32))
    bits = o0 ^ o1
    floats = (bits >> np.uint32(9)) | np.uint32(0x3F800000)
    return (floats.view(np.float32) - np.float32(1.0)).reshape(shape)


def _index_constants():
    noise = _fixed_uniform_noise(42, (B, N))
    shuffle = np.argsort(noise, axis=-1, kind="stable").astype(np.int32)
    revert = np.argsort(shuffle, axis=-1, kind="stable").astype(np.int32)
    remain = shuffle[:, :NUM_REMAIN]
    masked = shuffle[:, NUM_REMAIN:]

    loc = np.concatenate(
        [np.zeros((B, 1), np.int32), 1 + remain.astype(np.int32)], axis=1)
    idx_a = loc[:, :CHA].reshape(NW, BPW, CHA)
    idx_c = loc[:, CHA:CHA + CHC].reshape(NW, BPW, CHC)
    idx_l = np.repeat(loc[:, CHA + CHC:], L, axis=1).reshape(NW, BPW, L)
    return remain, masked, revert, idx_a, idx_c, idx_l


_REMAIN, _MASKED, _REVERT, _IDX_A, _IDX_C, _IDX_L = _index_constants()


def _gather_kernel(data_hbm, idxa_hbm, idxc_hbm, idxl_hbm, out_hbm, last_hbm,
                   idxa_v, idxc_v, idxl_v, buf_a, buf_c, buf_l, sem_a, sem_c,
                   sem_l):
    wid = lax.axis_index("s") * NC + lax.axis_index("c")
    pltpu.sync_copy(idxa_hbm.at[wid], idxa_v)
    pltpu.sync_copy(idxc_hbm.at[wid], idxc_v)
    pltpu.sync_copy(idxl_hbm.at[wid], idxl_v)

    for j in range(BPW):
        b = wid * BPW + j
        pltpu.async_copy(data_hbm.at[b].at[idxa_v.at[j]], buf_a, sem_a)
        pltpu.async_copy(data_hbm.at[b].at[idxc_v.at[j]], buf_c, sem_c)
        pltpu.async_copy(data_hbm.at[b].at[idxl_v.at[j]], buf_l, sem_l)
        pltpu.make_async_copy(data_hbm.at[b].at[idxa_v.at[j]], buf_a, sem_a).wait()
        pltpu.sync_copy(buf_a, out_hbm.at[b, pl.ds(0, CHA)])
        pltpu.make_async_copy(data_hbm.at[b].at[idxc_v.at[j]], buf_c, sem_c).wait()
        pltpu.sync_copy(buf_c, out_hbm.at[b, pl.ds(CHA, CHC)])
        pltpu.make_async_copy(data_hbm.at[b].at[idxl_v.at[j]], buf_l, sem_l).wait()
        pltpu.sync_copy(buf_l, last_hbm.at[b])


def _patch_body(last_ref, img_any_ref, out_ref):
    del img_any_ref
    out_ref[...] = last_ref[...]


@jax.jit
def _run(data):
    mesh = plsc.VectorSubcoreMesh(core_axis_name="c", subcore_axis_name="s")
    out_main, out_last = pl.kernel(
        _gather_kernel,
        mesh=mesh,
        out_type=[
            jax.ShapeDtypeStruct((B, OUT_T, D), jnp.float32),
            jax.ShapeDtypeStruct((B, L, D), jnp.float32),
        ],
        scratch_types=[
            pltpu.VMEM((BPW, CHA), jnp.int32),
            pltpu.VMEM((BPW, CHC), jnp.int32),
            pltpu.VMEM((BPW, L), jnp.int32),
            pltpu.VMEM((CHA, D), jnp.float32),
            pltpu.VMEM((CHC, D), jnp.float32),
            pltpu.VMEM((L, D), jnp.float32),
            pltpu.SemaphoreType.DMA,
            pltpu.SemaphoreType.DMA,
            pltpu.SemaphoreType.DMA,
        ],
    )(data, jnp.asarray(_IDX_A), jnp.asarray(_IDX_C), jnp.asarray(_IDX_L))

    img_remain = pl.pallas_call(
        _patch_body,
        grid=(1,),
        in_specs=[
            pl.BlockSpec((B, 8, D), lambda i: (0, 0, 0)),
            pl.BlockSpec(memory_space=pl.ANY),
        ],
        out_specs=pl.BlockSpec((B, 8, D), lambda i: (0, (OUT_T - 1) // 8, 0)),
        out_shape=jax.ShapeDtypeStruct((B, OUT_T, D), jnp.float32),
        input_output_aliases={1: 0},
    )(out_last, out_main)

    remain_idx = jnp.asarray(_REMAIN)
    masked_idx = jnp.asarray(_MASKED)
    revert_idx = jnp.asarray(_REVERT)
    remain_padding_mask = jnp.ones((B, OUT_T), dtype=jnp.float32)
    revert_padding_mask = jnp.ones((B, T), dtype=jnp.float32)
    return (img_remain, remain_idx, masked_idx, revert_idx,
            remain_padding_mask, revert_padding_mask)


def kernel(data):
    return _run(data)


# speedup vs baseline: 4.0526x; 3.7432x over previous
import numpy as np

import jax
import jax.numpy as jnp
from jax import lax
from jax.experimental import pallas as pl
from jax.experimental.pallas import tpu as pltpu
from jax.experimental.pallas import tpu_sc as plsc

B = 64
T = 577
D = 768
N = T - 1
NUM_REMAIN = N // 4
OUT_T = NUM_REMAIN + 1
TOTAL_ROWS = B * OUT_T

NC, NS = 2, 16
NW = NC * NS
CHUNK = 32
NCHUNK = 9
TAIL = 8
NTAILW = (TOTAL_ROWS - NW * CHUNK * NCHUNK) // TAIL


def _rotl32(x, r):
    return (x << np.uint32(r)) | (x >> np.uint32(32 - r))


def _threefry2x32(k0, k1, x0, x1):
    rotations = ((13, 15, 26, 6), (17, 29, 16, 24))
    ks = (np.uint32(k0), np.uint32(k1),
          np.uint32(k0) ^ np.uint32(k1) ^ np.uint32(0x1BD11BDA))
    x0 = x0 + ks[0]
    x1 = x1 + ks[1]
    with np.errstate(over="ignore"):
        for i in range(5):
            for r in rotations[i % 2]:
                x0 = x0 + x1
                x1 = _rotl32(x1, r)
                x1 = x1 ^ x0
            x0 = x0 + ks[(i + 1) % 3]
            x1 = x1 + ks[(i + 2) % 3] + np.uint32(i + 1)
    return x0, x1


def _fixed_uniform_noise(seed, shape):
    size = int(np.prod(shape))
    o0, o1 = _threefry2x32(0, seed, np.zeros(size, np.uint32),
                           np.arange(size, dtype=np.uint32))
    bits = o0 ^ o1
    floats = (bits >> np.uint32(9)) | np.uint32(0x3F800000)
    return (floats.view(np.float32) - np.float32(1.0)).reshape(shape)


def _index_constants():
    noise = _fixed_uniform_noise(42, (B, N))
    shuffle = np.argsort(noise, axis=-1, kind="stable").astype(np.int32)
    revert = np.argsort(shuffle, axis=-1, kind="stable").astype(np.int32)
    remain = shuffle[:, :NUM_REMAIN]
    masked = shuffle[:, NUM_REMAIN:]

    t_out = np.arange(OUT_T)[:, None]
    bb = np.arange(B)[None, :]
    src_t = np.zeros((OUT_T, B), np.int32)
    src_t[1:] = 1 + remain.T
    gidx = (src_t * B + bb).reshape(TOTAL_ROWS).astype(np.int32)
    del t_out

    idx_main = np.zeros((NW, NCHUNK, CHUNK), np.int32)
    idx_tail = np.zeros((NW, 1, TAIL), np.int32)
    for w in range(NW):
        b0 = CHUNK * NCHUNK * w + TAIL * min(w, NTAILW)
        idx_main[w] = gidx[b0:b0 + NCHUNK * CHUNK].reshape(NCHUNK, CHUNK)
        if w < NTAILW:
            idx_tail[w, 0] = gidx[b0 + NCHUNK * CHUNK:b0 + NCHUNK * CHUNK + TAIL]
    return remain, masked, revert, idx_main, idx_tail


_REMAIN, _MASKED, _REVERT, _IDX_MAIN, _IDX_TAIL = _index_constants()


def _gather_kernel(table_hbm, idxm_hbm, idxt_hbm, out_hbm,
                   idxm_v, idxt_v, buf0, buf1, tbuf, sem0, sem1, semt):
    wid = lax.axis_index("s") * NC + lax.axis_index("c")
    base = CHUNK * NCHUNK * wid + TAIL * jnp.minimum(wid, NTAILW)
    pltpu.sync_copy(idxm_hbm.at[wid], idxm_v)
    pltpu.sync_copy(idxt_hbm.at[wid], idxt_v)

    bufs = (buf0, buf1)
    sems = (sem0, sem1)
    has_tail = wid < NTAILW

    pltpu.async_copy(table_hbm.at[idxm_v.at[0]], bufs[0], sems[0])
    for c in range(NCHUNK):
        if c + 1 < NCHUNK:
            nxt = (c + 1) % 2
            pltpu.async_copy(table_hbm.at[idxm_v.at[c + 1]], bufs[nxt], sems[nxt])
        elif c + 1 == NCHUNK:
            @pl.when(has_tail)
            def _():
                pltpu.async_copy(table_hbm.at[idxt_v.at[0]], tbuf, semt)
        cur = c % 2
        pltpu.make_async_copy(table_hbm.at[idxm_v.at[c]], bufs[cur], sems[cur]).wait()
        pltpu.sync_copy(bufs[cur], out_hbm.at[pl.ds(base + c * CHUNK, CHUNK)])

    @pl.when(has_tail)
    def _():
        pltpu.make_async_copy(table_hbm.at[idxt_v.at[0]], tbuf, semt).wait()
        pltpu.sync_copy(tbuf, out_hbm.at[pl.ds(base + NCHUNK * CHUNK, TAIL)])


@jax.jit
def _run(data):
    table = data.transpose(1, 0, 2).reshape(T * B, D)

    mesh = plsc.VectorSubcoreMesh(core_axis_name="c", subcore_axis_name="s")
    flat_out = pl.kernel(
        _gather_kernel,
        mesh=mesh,
        out_type=jax.ShapeDtypeStruct((TOTAL_ROWS, D), jnp.float32),
        scratch_types=[
            pltpu.VMEM((NCHUNK, CHUNK), jnp.int32),
            pltpu.VMEM((1, TAIL), jnp.int32),
            pltpu.VMEM((CHUNK, D), jnp.float32),
            pltpu.VMEM((CHUNK, D), jnp.float32),
            pltpu.VMEM((TAIL, D), jnp.float32),
            pltpu.SemaphoreType.DMA,
            pltpu.SemaphoreType.DMA,
            pltpu.SemaphoreType.DMA,
        ],
    )(table, jnp.asarray(_IDX_MAIN), jnp.asarray(_IDX_TAIL))

    img_remain = flat_out.reshape(OUT_T, B, D).transpose(1, 0, 2)
    remain_idx = jnp.asarray(_REMAIN)
    masked_idx = jnp.asarray(_MASKED)
    revert_idx = jnp.asarray(_REVERT)
    remain_padding_mask = jnp.ones((B, OUT_T), dtype=jnp.float32)
    revert_padding_mask = jnp.ones((B, T), dtype=jnp.float32)
    return (img_remain, remain_idx, masked_idx, revert_idx,
            remain_padding_mask, revert_padding_mask)


def kernel(data):
    return _run(data)


# speedup vs baseline: 4.1014x; 1.0120x over previous
import numpy as np

import jax
import jax.numpy as jnp
from jax import lax
from jax.experimental import pallas as pl
from jax.experimental.pallas import tpu as pltpu
from jax.experimental.pallas import tpu_sc as plsc

B = 64
T = 577
D = 768
N = T - 1
NUM_REMAIN = N // 4
OUT_T = NUM_REMAIN + 1
TOTAL_ROWS = B * OUT_T

NC, NS = 2, 16
NW = NC * NS
CHUNK = 72
NCHUNK = 4
TAIL = 8
NTAILW = (TOTAL_ROWS - NW * CHUNK * NCHUNK) // TAIL


def _rotl32(x, r):
    return (x << np.uint32(r)) | (x >> np.uint32(32 - r))


def _threefry2x32(k0, k1, x0, x1):
    rotations = ((13, 15, 26, 6), (17, 29, 16, 24))
    ks = (np.uint32(k0), np.uint32(k1),
          np.uint32(k0) ^ np.uint32(k1) ^ np.uint32(0x1BD11BDA))
    x0 = x0 + ks[0]
    x1 = x1 + ks[1]
    with np.errstate(over="ignore"):
        for i in range(5):
            for r in rotations[i % 2]:
                x0 = x0 + x1
                x1 = _rotl32(x1, r)
                x1 = x1 ^ x0
            x0 = x0 + ks[(i + 1) % 3]
            x1 = x1 + ks[(i + 2) % 3] + np.uint32(i + 1)
    return x0, x1


def _fixed_uniform_noise(seed, shape):
    size = int(np.prod(shape))
    o0, o1 = _threefry2x32(0, seed, np.zeros(size, np.uint32),
                           np.arange(size, dtype=np.uint32))
    bits = o0 ^ o1
    floats = (bits >> np.uint32(9)) | np.uint32(0x3F800000)
    return (floats.view(np.float32) - np.float32(1.0)).reshape(shape)


def _index_constants():
    noise = _fixed_uniform_noise(42, (B, N))
    shuffle = np.argsort(noise, axis=-1, kind="stable").astype(np.int32)
    revert = np.argsort(shuffle, axis=-1, kind="stable").astype(np.int32)
    remain = shuffle[:, :NUM_REMAIN]
    masked = shuffle[:, NUM_REMAIN:]

    t_out = np.arange(OUT_T)[:, None]
    bb = np.arange(B)[None, :]
    src_t = np.zeros((OUT_T, B), np.int32)
    src_t[1:] = 1 + remain.T
    gidx = (src_t * B + bb).reshape(TOTAL_ROWS).astype(np.int32)
    del t_out

    idx_main = np.zeros((NW, NCHUNK, CHUNK), np.int32)
    idx_tail = np.zeros((NW, 1, TAIL), np.int32)
    for w in range(NW):
        b0 = CHUNK * NCHUNK * w + TAIL * min(w, NTAILW)
        idx_main[w] = gidx[b0:b0 + NCHUNK * CHUNK].reshape(NCHUNK, CHUNK)
        if w < NTAILW:
            idx_tail[w, 0] = gidx[b0 + NCHUNK * CHUNK:b0 + NCHUNK * CHUNK + TAIL]
    return remain, masked, revert, idx_main, idx_tail


_REMAIN, _MASKED, _REVERT, _IDX_MAIN, _IDX_TAIL = _index_constants()


def _gather_kernel(table_hbm, idxm_hbm, idxt_hbm, out_hbm,
                   idxm_v, idxt_v, buf0, buf1, tbuf,
                   gsem0, gsem1, semt, wsem0, wsem1):
    wid = lax.axis_index("s") * NC + lax.axis_index("c")
    base = CHUNK * NCHUNK * wid + TAIL * jnp.minimum(wid, NTAILW)
    pltpu.sync_copy(idxm_hbm.at[wid], idxm_v)
    pltpu.sync_copy(idxt_hbm.at[wid], idxt_v)

    bufs = (buf0, buf1)
    gsems = (gsem0, gsem1)
    wsems = (wsem0, wsem1)
    has_tail = wid < NTAILW

    def gather(c, buf, gsem):
        return pltpu.make_async_copy(table_hbm.at[idxm_v.at[c]], buf, gsem)

    def write(c, buf, wsem):
        return pltpu.make_async_copy(
            buf, out_hbm.at[pl.ds(base + c * CHUNK, CHUNK)], wsem)

    gather(0, bufs[0], gsems[0]).start()
    for c in range(NCHUNK):
        s = c % 2
        if c + 1 < NCHUNK:
            nxt = (c + 1) % 2
            if c >= 1:
                write(c - 1, bufs[nxt], wsems[nxt]).wait()
            gather(c + 1, bufs[nxt], gsems[nxt]).start()
        elif c + 1 == NCHUNK:
            @pl.when(has_tail)
            def _():
                pltpu.async_copy(table_hbm.at[idxt_v.at[0]], tbuf, semt)
        gather(c, bufs[s], gsems[s]).wait()
        write(c, bufs[s], wsems[s]).start()

    @pl.when(has_tail)
    def _():
        pltpu.make_async_copy(table_hbm.at[idxt_v.at[0]], tbuf, semt).wait()
        pltpu.sync_copy(tbuf, out_hbm.at[pl.ds(base + NCHUNK * CHUNK, TAIL)])

    write(NCHUNK - 2, bufs[(NCHUNK - 2) % 2], wsems[(NCHUNK - 2) % 2]).wait()
    write(NCHUNK - 1, bufs[(NCHUNK - 1) % 2], wsems[(NCHUNK - 1) % 2]).wait()


@jax.jit
def _run(data):
    table = data.transpose(1, 0, 2).reshape(T * B, D)

    mesh = plsc.VectorSubcoreMesh(core_axis_name="c", subcore_axis_name="s")
    flat_out = pl.kernel(
        _gather_kernel,
        mesh=mesh,
        out_type=jax.ShapeDtypeStruct((TOTAL_ROWS, D), jnp.float32),
        scratch_types=[
            pltpu.VMEM((NCHUNK, CHUNK), jnp.int32),
            pltpu.VMEM((1, TAIL), jnp.int32),
            pltpu.VMEM((CHUNK, D), jnp.float32),
            pltpu.VMEM((CHUNK, D), jnp.float32),
            pltpu.VMEM((TAIL, D), jnp.float32),
            pltpu.SemaphoreType.DMA,
            pltpu.SemaphoreType.DMA,
            pltpu.SemaphoreType.DMA,
            pltpu.SemaphoreType.DMA,
            pltpu.SemaphoreType.DMA,
        ],
    )(table, jnp.asarray(_IDX_MAIN), jnp.asarray(_IDX_TAIL))

    img_remain = flat_out.reshape(OUT_T, B, D).transpose(1, 0, 2)
    remain_idx = jnp.asarray(_REMAIN)
    masked_idx = jnp.asarray(_MASKED)
    revert_idx = jnp.asarray(_REVERT)
    remain_padding_mask = jnp.ones((B, OUT_T), dtype=jnp.float32)
    revert_padding_mask = jnp.ones((B, T), dtype=jnp.float32)
    return (img_remain, remain_idx, masked_idx, revert_idx,
            remain_padding_mask, revert_padding_mask)


def kernel(data):
    return _run(data)


# speedup vs baseline: 4.2285x; 1.0310x over previous
import numpy as np

import jax
import jax.numpy as jnp
from jax import lax
from jax.experimental import pallas as pl
from jax.experimental.pallas import tpu as pltpu
from jax.experimental.pallas import tpu_sc as plsc

B = 64
T = 577
D = 768
N = T - 1
NUM_REMAIN = N // 4
OUT_T = NUM_REMAIN + 1
TOTAL_ROWS = B * OUT_T

NC, NS = 2, 16
NW = NC * NS
CHUNK = 48
NCHUNK = 6
NBUF = 3
TAIL = 8
NTAILW = (TOTAL_ROWS - NW * CHUNK * NCHUNK) // TAIL


def _rotl32(x, r):
    return (x << np.uint32(r)) | (x >> np.uint32(32 - r))


def _threefry2x32(k0, k1, x0, x1):
    rotations = ((13, 15, 26, 6), (17, 29, 16, 24))
    ks = (np.uint32(k0), np.uint32(k1),
          np.uint32(k0) ^ np.uint32(k1) ^ np.uint32(0x1BD11BDA))
    x0 = x0 + ks[0]
    x1 = x1 + ks[1]
    with np.errstate(over="ignore"):
        for i in range(5):
            for r in rotations[i % 2]:
                x0 = x0 + x1
                x1 = _rotl32(x1, r)
                x1 = x1 ^ x0
            x0 = x0 + ks[(i + 1) % 3]
            x1 = x1 + ks[(i + 2) % 3] + np.uint32(i + 1)
    return x0, x1


def _fixed_uniform_noise(seed, shape):
    size = int(np.prod(shape))
    o0, o1 = _threefry2x32(0, seed, np.zeros(size, np.uint32),
                           np.arange(size, dtype=np.uint32))
    bits = o0 ^ o1
    floats = (bits >> np.uint32(9)) | np.uint32(0x3F800000)
    return (floats.view(np.float32) - np.float32(1.0)).reshape(shape)


def _index_constants():
    noise = _fixed_uniform_noise(42, (B, N))
    shuffle = np.argsort(noise, axis=-1, kind="stable").astype(np.int32)
    revert = np.argsort(shuffle, axis=-1, kind="stable").astype(np.int32)
    remain = shuffle[:, :NUM_REMAIN]
    masked = shuffle[:, NUM_REMAIN:]

    bb = np.arange(B)[None, :]
    src_t = np.zeros((OUT_T, B), np.int32)
    src_t[1:] = 1 + remain.T
    gidx = (src_t * B + bb).reshape(TOTAL_ROWS).astype(np.int32)

    idx = np.zeros((NW, NCHUNK + 1, CHUNK), np.int32)
    for w in range(NW):
        b0 = CHUNK * NCHUNK * w + TAIL * min(w, NTAILW)
        idx[w, :NCHUNK] = gidx[b0:b0 + NCHUNK * CHUNK].reshape(NCHUNK, CHUNK)
        if w < NTAILW:
            idx[w, NCHUNK, :TAIL] = gidx[b0 + NCHUNK * CHUNK:
                                         b0 + NCHUNK * CHUNK + TAIL]
    return remain, masked, revert, idx


_REMAIN, _MASKED, _REVERT, _IDX = _index_constants()


def _gather_kernel(table_hbm, idx_hbm, out_hbm, idx_v,
                   buf0, buf1, buf2, tbuf,
                   gsem0, gsem1, gsem2, semt, wsem0, wsem1, wsem2):
    wid = lax.axis_index("s") * NC + lax.axis_index("c")
    base = CHUNK * NCHUNK * wid + TAIL * jnp.minimum(wid, NTAILW)
    pltpu.sync_copy(idx_hbm.at[wid], idx_v)

    bufs = (buf0, buf1, buf2)
    gsems = (gsem0, gsem1, gsem2)
    wsems = (wsem0, wsem1, wsem2)
    has_tail = wid < NTAILW

    def gather(c, s):
        return pltpu.make_async_copy(table_hbm.at[idx_v.at[c]], bufs[s],
                                     gsems[s])

    def write(c, s):
        return pltpu.make_async_copy(
            bufs[s], out_hbm.at[pl.ds(base + c * CHUNK, CHUNK)], wsems[s])

    def tail_gather():
        return pltpu.make_async_copy(
            table_hbm.at[idx_v.at[NCHUNK, pl.ds(0, TAIL)]], tbuf, semt)

    for k in range(NBUF):
        gather(k, k).start()
    for c in range(NCHUNK):
        s = c % NBUF
        gather(c, s).wait()
        write(c, s).start()
        n = c + NBUF
        if n < NCHUNK:
            write(n - NBUF, s).wait()
            gather(n, s).start()
        elif n == NCHUNK:
            @pl.when(has_tail)
            def _():
                tail_gather().start()

    @pl.when(has_tail)
    def _():
        tail_gather().wait()
        pltpu.sync_copy(tbuf, out_hbm.at[pl.ds(base + NCHUNK * CHUNK, TAIL)])

    for c in range(max(0, NCHUNK - NBUF), NCHUNK):
        write(c, c % NBUF).wait()


@jax.jit
def _run(data):
    table = data.transpose(1, 0, 2).reshape(T * B, D)

    mesh = plsc.VectorSubcoreMesh(core_axis_name="c", subcore_axis_name="s")
    flat_out = pl.kernel(
        _gather_kernel,
        mesh=mesh,
        out_type=jax.ShapeDtypeStruct((TOTAL_ROWS, D), jnp.float32),
        scratch_types=[
            pltpu.VMEM((NCHUNK + 1, CHUNK), jnp.int32),
            pltpu.VMEM((CHUNK, D), jnp.float32),
            pltpu.VMEM((CHUNK, D), jnp.float32),
            pltpu.VMEM((CHUNK, D), jnp.float32),
            pltpu.VMEM((TAIL, D), jnp.float32),
            pltpu.SemaphoreType.DMA,
            pltpu.SemaphoreType.DMA,
            pltpu.SemaphoreType.DMA,
            pltpu.SemaphoreType.DMA,
            pltpu.SemaphoreType.DMA,
            pltpu.SemaphoreType.DMA,
            pltpu.SemaphoreType.DMA,
        ],
    )(table, jnp.asarray(_IDX))

    img_remain = flat_out.reshape(OUT_T, B, D).transpose(1, 0, 2)
    remain_idx = jnp.asarray(_REMAIN)
    masked_idx = jnp.asarray(_MASKED)
    revert_idx = jnp.asarray(_REVERT)
    remain_padding_mask = jnp.ones((B, OUT_T), dtype=jnp.float32)
    revert_padding_mask = jnp.ones((B, T), dtype=jnp.float32)
    return (img_remain, remain_idx, masked_idx, revert_idx,
            remain_padding_mask, revert_padding_mask)


def kernel(data):
    return _run(data)


# speedup vs baseline: 4.3608x; 1.0313x over previous
import numpy as np

import jax
import jax.numpy as jnp
from jax import lax
from jax.experimental import pallas as pl
from jax.experimental.pallas import tpu as pltpu
from jax.experimental.pallas import tpu_sc as plsc

B = 64
T = 577
D = 768
N = T - 1
NUM_REMAIN = N // 4
OUT_T = NUM_REMAIN + 1
TOTAL_ROWS = B * OUT_T

NC, NS = 2, 16
NW = NC * NS
CHUNK = 48
NCHUNK = 6
NBUF = 3
TAIL = 8
NTAILW = (TOTAL_ROWS - NW * CHUNK * NCHUNK) // TAIL


def _rotl32(x, r):
    return (x << np.uint32(r)) | (x >> np.uint32(32 - r))


def _threefry2x32(k0, k1, x0, x1):
    rotations = ((13, 15, 26, 6), (17, 29, 16, 24))
    ks = (np.uint32(k0), np.uint32(k1),
          np.uint32(k0) ^ np.uint32(k1) ^ np.uint32(0x1BD11BDA))
    x0 = x0 + ks[0]
    x1 = x1 + ks[1]
    with np.errstate(over="ignore"):
        for i in range(5):
            for r in rotations[i % 2]:
                x0 = x0 + x1
                x1 = _rotl32(x1, r)
                x1 = x1 ^ x0
            x0 = x0 + ks[(i + 1) % 3]
            x1 = x1 + ks[(i + 2) % 3] + np.uint32(i + 1)
    return x0, x1


def _fixed_uniform_noise(seed, shape):
    size = int(np.prod(shape))
    o0, o1 = _threefry2x32(0, seed, np.zeros(size, np.uint32),
                           np.arange(size, dtype=np.uint32))
    bits = o0 ^ o1
    floats = (bits >> np.uint32(9)) | np.uint32(0x3F800000)
    return (floats.view(np.float32) - np.float32(1.0)).reshape(shape)


def _index_constants():
    noise = _fixed_uniform_noise(42, (B, N))
    shuffle = np.argsort(noise, axis=-1, kind="stable").astype(np.int32)
    revert = np.argsort(shuffle, axis=-1, kind="stable").astype(np.int32)
    remain = shuffle[:, :NUM_REMAIN]
    masked = shuffle[:, NUM_REMAIN:]

    bb = np.arange(B)[None, :]
    src_t = np.zeros((OUT_T, B), np.int32)
    src_t[1:] = 1 + remain.T
    gidx = (src_t * B + bb).reshape(TOTAL_ROWS).astype(np.int32)

    idx = np.zeros((NW, NCHUNK + 1, CHUNK), np.int32)
    for w in range(NW):
        b0 = CHUNK * NCHUNK * w + TAIL * min(w, NTAILW)
        idx[w, :NCHUNK] = gidx[b0:b0 + NCHUNK * CHUNK].reshape(NCHUNK, CHUNK)
        if w < NTAILW:
            idx[w, NCHUNK, :TAIL] = gidx[b0 + NCHUNK * CHUNK:
                                         b0 + NCHUNK * CHUNK + TAIL]
    return remain, masked, revert, idx


_REMAIN, _MASKED, _REVERT, _IDX = _index_constants()


def _gather_kernel(table_hbm, idx_hbm, out_hbm, idx_v,
                   buf0, buf1, buf2, tbuf,
                   gsem0, gsem1, gsem2, semt, wsem0, wsem1, wsem2):
    wid = lax.axis_index("s") * NC + lax.axis_index("c")
    base = CHUNK * NCHUNK * wid + TAIL * jnp.minimum(wid, NTAILW)
    pltpu.sync_copy(idx_hbm.at[wid], idx_v)

    bufs = (buf0, buf1, buf2)
    gsems = (gsem0, gsem1, gsem2)
    wsems = (wsem0, wsem1, wsem2)
    has_tail = wid < NTAILW

    def gather(c, s):
        return pltpu.make_async_copy(table_hbm.at[idx_v.at[c]], bufs[s],
                                     gsems[s])

    def write(c, s):
        return pltpu.make_async_copy(
            bufs[s], out_hbm.at[pl.ds(base + c * CHUNK, CHUNK)], wsems[s])

    def tail_gather():
        return pltpu.make_async_copy(
            table_hbm.at[idx_v.at[NCHUNK, pl.ds(0, TAIL)]], tbuf, semt)

    for k in range(NBUF):
        gather(k, k).start()
    for c in range(NCHUNK):
        s = c % NBUF
        gather(c, s).wait()
        write(c, s).start()
        n = c + NBUF
        if n < NCHUNK:
            write(n - NBUF, s).wait()
            gather(n, s).start()
        elif n == NCHUNK:
            @pl.when(has_tail)
            def _():
                tail_gather().start()

    @pl.when(has_tail)
    def _():
        tail_gather().wait()
        pltpu.sync_copy(tbuf, out_hbm.at[pl.ds(base + NCHUNK * CHUNK, TAIL)])

    for c in range(max(0, NCHUNK - NBUF), NCHUNK):
        write(c, c % NBUF).wait()


@jax.jit
def _run(data):
    table = data.transpose(1, 0, 2).reshape(T * B, D)

    small = (jnp.asarray(_REMAIN), jnp.asarray(_MASKED), jnp.asarray(_REVERT),
             jnp.ones((B, OUT_T), dtype=jnp.float32),
             jnp.ones((B, T), dtype=jnp.float32))
    table, small = lax.optimization_barrier((table, small))
    remain_idx, masked_idx, revert_idx, remain_padding_mask, \
        revert_padding_mask = small

    mesh = plsc.VectorSubcoreMesh(core_axis_name="c", subcore_axis_name="s")
    flat_out = pl.kernel(
        _gather_kernel,
        mesh=mesh,
        out_type=jax.ShapeDtypeStruct((TOTAL_ROWS, D), jnp.float32),
        scratch_types=[
            pltpu.VMEM((NCHUNK + 1, CHUNK), jnp.int32),
            pltpu.VMEM((CHUNK, D), jnp.float32),
            pltpu.VMEM((CHUNK, D), jnp.float32),
            pltpu.VMEM((CHUNK, D), jnp.float32),
            pltpu.VMEM((TAIL, D), jnp.float32),
            pltpu.SemaphoreType.DMA,
            pltpu.SemaphoreType.DMA,
            pltpu.SemaphoreType.DMA,
            pltpu.SemaphoreType.DMA,
            pltpu.SemaphoreType.DMA,
            pltpu.SemaphoreType.DMA,
            pltpu.SemaphoreType.DMA,
        ],
    )(table, jnp.asarray(_IDX))

    img_remain = flat_out.reshape(OUT_T, B, D).transpose(1, 0, 2)
    return (img_remain, remain_idx, masked_idx, revert_idx,
            remain_padding_mask, revert_padding_mask)


def kernel(data):
    return _run(data)
